# baseline (speedup 1.0000x reference)
import functools

import jax
import jax.numpy as jnp
import numpy as np
from jax.experimental import pallas as pl

N = 4096
D = 512
H = 128
OUT = 512
NOISE = 5

_BLK = 256
_BLK2 = 512
_PRE_BLK = 512

_bf16 = jnp.bfloat16
_f32 = jnp.float32


def _dot(a, b):
    return jax.lax.dot_general(
        a, b, (((1,), (0,)), ((), ())), preferred_element_type=_f32
    )


_U_LO = np.float32(-0.99999994)
_U_SPAN = np.float32(1.99999994)
_SQRT2 = np.float32(1.4142135623730951)


def _normal_from_bits(bits):
    u01 = jax.lax.bitcast_convert_type(
        (bits >> 9) | np.uint32(0x3F800000), _f32) - 1.0
    u = jnp.maximum(_U_LO, u01 * _U_SPAN + _U_LO)
    w = -jnp.log1p(-u * u)
    ws = w - 2.5
    ps = np.float32(2.81022636e-08)
    for c in (3.43273939e-07, -3.5233877e-06, -4.39150654e-06, 0.00021858087,
              -0.00125372503, -0.00417768164, 0.246640727, 1.50140941):
        ps = np.float32(c) + ps * ws
    wb = jnp.sqrt(w) - 3.0
    pb = np.float32(-0.000200214257)
    for c in (0.000100950558, 0.00134934322, -0.00367342844, 0.00573950773,
              -0.0076224613, 0.00943887047, 1.00167406, 2.83297682):
        pb = np.float32(c) + pb * wb
    return _SQRT2 * jnp.where(w < 5.0, ps, pb) * u


def _pre_kernel(x_ref, nf_ref, w1x_ref, w1n_ref, wv1_ref, b1_ref, bv1_ref, p_ref):
    xb = x_ref[...].astype(_bf16)
    w1n = w1n_ref[...].astype(_bf16)
    g_mu = _dot(xb, w1x_ref[...].astype(_bf16))
    g_lv = _dot(xb, wv1_ref[...].astype(_bf16))
    n0 = _dot(nf_ref[:, 0:NOISE].astype(_bf16), w1n)
    n1 = _dot(nf_ref[:, NOISE:2 * NOISE].astype(_bf16), w1n)
    b1 = b1_ref[...]
    bv1 = bv1_ref[...]
    p0 = g_mu + n0 + b1
    p1 = g_mu + n1 + b1
    pv = g_lv + bv1
    p_ref[...] = jnp.concatenate([p0, p1, pv], axis=1).astype(_bf16)


def _pre(x, nf, w1x, w1n, wv1, b1, bv1):
    grid = N // _PRE_BLK
    return pl.pallas_call(
        _pre_kernel,
        grid=(grid,),
        in_specs=[
            pl.BlockSpec((_PRE_BLK, D), lambda i: (i, 0)),
            pl.BlockSpec((_PRE_BLK, 2 * NOISE), lambda i: (i, 0)),
            pl.BlockSpec((D, H), lambda i: (0, 0)),
            pl.BlockSpec((NOISE, H), lambda i: (0, 0)),
            pl.BlockSpec((D, H), lambda i: (0, 0)),
            pl.BlockSpec((1, H), lambda i: (0, 0)),
            pl.BlockSpec((1, H), lambda i: (0, 0)),
        ],
        out_specs=pl.BlockSpec((_PRE_BLK, 3 * H), lambda i: (i, 0)),
        out_shape=jax.ShapeDtypeStruct((N, 3 * H), _bf16),
    )(x, nf, w1x, w1n, wv1, b1, bv1)


def _prop1_kernel(adj_ref, p_ref, w2_ref, wv2_ref, b2_ref, bv2_ref, q_ref):
    t1 = _dot(adj_ref[...].astype(_bf16), p_ref[...])
    h = jax.nn.relu(t1).astype(_bf16)
    w2 = w2_ref[...].astype(_bf16)
    q0 = _dot(h[:, 0:H], w2) + b2_ref[...]
    q1 = _dot(h[:, H:2 * H], w2) + b2_ref[...]
    qv = _dot(h[:, 2 * H:3 * H], wv2_ref[...].astype(_bf16)) + bv2_ref[...]
    q_ref[...] = jnp.concatenate([q0, q1, qv], axis=1).astype(_bf16)


def _prop1(adj, p, w2, wv2, b2, bv2):
    grid = N // _BLK
    return pl.pallas_call(
        _prop1_kernel,
        grid=(grid,),
        in_specs=[
            pl.BlockSpec((_BLK, N), lambda i: (i, 0)),
            pl.BlockSpec((N, 3 * H), lambda i: (0, 0)),
            pl.BlockSpec((H, OUT), lambda i: (0, 0)),
            pl.BlockSpec((H, OUT), lambda i: (0, 0)),
            pl.BlockSpec((1, OUT), lambda i: (0, 0)),
            pl.BlockSpec((1, OUT), lambda i: (0, 0)),
        ],
        out_specs=pl.BlockSpec((_BLK, 3 * OUT), lambda i: (i, 0)),
        out_shape=jax.ShapeDtypeStruct((N, 3 * OUT), _bf16),
    )(adj, p, w2, wv2, b2, bv2)


def _prop2_kernel(adj_ref, q_ref, eps_ref, mm_ref, ms_ref, mz_ref,
                  lv_ref, z_ref, mv_ref, zb_ref):
    t2 = _dot(adj_ref[...].astype(_bf16), q_ref[...])
    mu_iw = t2[:, 0:OUT]
    mu_star = t2[:, OUT:2 * OUT]
    logv = t2[:, 2 * OUT:3 * OUT]
    sigma = jnp.exp(0.5 * logv)
    z = mu_iw + sigma * _normal_from_bits(eps_ref[...])
    mm_ref[:, 0, :] = mu_star
    mm_ref[:, 1, :] = mu_iw
    ms_ref[:, 0, :] = sigma
    ms_ref[:, 1, :] = sigma
    mz_ref[:, 0, :] = z
    mz_ref[:, 1, :] = z
    lv_ref[...] = logv
    z_ref[...] = z
    mv_ref[...] = mu_iw
    zb_ref[...] = z.astype(_bf16)


def _prop2(adj, q, eps):
    grid = N // _BLK2
    wide = jax.ShapeDtypeStruct((N, 1, 2, OUT), _f32)
    slim3 = jax.ShapeDtypeStruct((N, 1, OUT), _f32)
    wspec = pl.BlockSpec((_BLK2, None, 2, OUT), lambda i: (i, 0, 0, 0))
    sspec3 = pl.BlockSpec((_BLK2, None, OUT), lambda i: (i, 0, 0))
    sspec = pl.BlockSpec((_BLK2, OUT), lambda i: (i, 0))
    return pl.pallas_call(
        _prop2_kernel,
        grid=(grid,),
        in_specs=[
            pl.BlockSpec((_BLK2, N), lambda i: (i, 0)),
            pl.BlockSpec((N, 3 * OUT), lambda i: (0, 0)),
            sspec,
        ],
        out_specs=[wspec, wspec, wspec, sspec3, sspec3, sspec, sspec],
        out_shape=[wide, wide, wide, slim3, slim3,
                   jax.ShapeDtypeStruct((N, OUT), _f32),
                   jax.ShapeDtypeStruct((N, OUT), _bf16)],
    )(adj, q, eps)


def _attr_kernel(x_ref, nfa_ref, wax_ref, wan_ref, wavx_ref, ba1_ref, bav1_ref,
                 wa2_ref, wav2_ref, ba2_ref, bav2_ref, epsa_ref,
                 mm_ref, ms_ref, mz_ref, lv_ref, z_ref, mv_ref, zb_ref):
    xb = x_ref[...].astype(_bf16)
    tdot = lambda a, b: jax.lax.dot_general(
        a, b, (((0,), (0,)), ((), ())), preferred_element_type=_f32)
    wan = wan_ref[...].astype(_bf16)
    g = tdot(xb, wax_ref[...].astype(_bf16))
    gv = tdot(xb, wavx_ref[...].astype(_bf16))
    n0 = _dot(nfa_ref[:, 0:NOISE].astype(_bf16), wan)
    n1 = _dot(nfa_ref[:, NOISE:2 * NOISE].astype(_bf16), wan)
    ba1 = ba1_ref[...]
    h0 = jax.nn.relu(g + n0 + ba1).astype(_bf16)
    h1 = jax.nn.relu(g + n1 + ba1).astype(_bf16)
    hv = jax.nn.relu(gv + bav1_ref[...]).astype(_bf16)
    wa2 = wa2_ref[...].astype(_bf16)
    mu_iw = _dot(h0, wa2) + ba2_ref[...]
    mu_star = _dot(h1, wa2) + ba2_ref[...]
    logv = _dot(hv, wav2_ref[...].astype(_bf16)) + bav2_ref[...]
    sigma = jnp.exp(0.5 * logv)
    z = mu_iw + sigma * _normal_from_bits(epsa_ref[...])
    mm_ref[:, 0, :] = mu_star
    mm_ref[:, 1, :] = mu_iw
    ms_ref[:, 0, :] = sigma
    ms_ref[:, 1, :] = sigma
    mz_ref[:, 0, :] = z
    mz_ref[:, 1, :] = z
    lv_ref[...] = logv
    z_ref[...] = z
    mv_ref[...] = mu_iw
    zb_ref[...] = z.astype(_bf16)


def _attr(x, nfa, wax, wan, wavx, ba1, bav1, wa2, wav2, ba2, bav2, epsa):
    a = D
    full = lambda r, c: pl.BlockSpec((r, c), lambda i: (0, 0))
    full3 = lambda r, c: pl.BlockSpec((r, None, c), lambda i: (0, 0, 0))
    full4 = lambda r, c: pl.BlockSpec((r, None, 2, c), lambda i: (0, 0, 0, 0))
    wide = jax.ShapeDtypeStruct((a, 1, 2, OUT), _f32)
    slim3 = jax.ShapeDtypeStruct((a, 1, OUT), _f32)
    return pl.pallas_call(
        _attr_kernel,
        grid=(1,),
        in_specs=[
            full(N, a), full(a, 2 * NOISE), full(N, H), full(NOISE, H),
            full(N, H), full(1, H), full(1, H), full(H, OUT), full(H, OUT),
            full(1, OUT), full(1, OUT), full(a, OUT),
        ],
        out_specs=[full4(a, OUT), full4(a, OUT), full4(a, OUT),
                   full3(a, OUT), full3(a, OUT), full(a, OUT), full(a, OUT)],
        out_shape=[wide, wide, wide, slim3, slim3,
                   jax.ShapeDtypeStruct((a, OUT), _f32),
                   jax.ShapeDtypeStruct((a, OUT), _bf16)],
    )(x, nfa, wax, wan, wavx, ba1, bav1, wa2, wav2, ba2, bav2, epsa)


def _decode_kernel(zu_ref, zall_ref, za_ref, lu_ref, la_ref):
    zu = zu_ref[...]
    bt_dot = lambda a, b: jax.lax.dot_general(
        a, b, (((1,), (1,)), ((), ())), preferred_element_type=_f32)
    lu_ref[...] = bt_dot(zu, zall_ref[...])
    la_ref[...] = bt_dot(zu, za_ref[...])


def _decode(zu_bf, za_bf):
    grid = N // _BLK2
    return pl.pallas_call(
        _decode_kernel,
        grid=(grid,),
        in_specs=[
            pl.BlockSpec((_BLK2, OUT), lambda i: (i, 0)),
            pl.BlockSpec((N, OUT), lambda i: (0, 0)),
            pl.BlockSpec((D, OUT), lambda i: (0, 0)),
        ],
        out_specs=[
            pl.BlockSpec((_BLK2, None, N), lambda i: (i, 0, 0)),
            pl.BlockSpec((_BLK2, None, D), lambda i: (i, 0, 0)),
        ],
        out_shape=[
            jax.ShapeDtypeStruct((N, 1, N), _f32),
            jax.ShapeDtypeStruct((N, 1, D), _f32),
        ],
    )(zu_bf, zu_bf, za_bf)


def kernel(adj, x, Wnm1, bnm1, Wnm2, bnm2, Wnv1, bnv1, Wnv2, bnv2,
           Wam1, bam1, Wam2, bam2, Wav1, bav1, Wav2, bav2):
    n, a = x.shape
    key = jax.random.key(42)
    k1, k2, k3, k4 = jax.random.split(key, 4)
    nf = jax.random.bernoulli(k1, 0.5, (n, 2 * NOISE)).astype(_f32)
    nfa = jax.random.bernoulli(k3, 0.5, (a, 2 * NOISE)).astype(_f32)
    eps_u = jax.random.bits(k2, (n, OUT), jnp.uint32)
    eps_a = jax.random.bits(k4, (a, OUT), jnp.uint32)

    w1x = Wnm1[NOISE:]
    w1n = Wnm1[:NOISE]
    wv1 = Wnv1
    w2 = Wnm2
    wv2 = Wnv2
    b1 = bnm1.reshape(1, H)
    bv1 = bnv1.reshape(1, H)
    b2 = bnm2.reshape(1, OUT)
    bv2 = bnv2.reshape(1, OUT)

    p = _pre(x, nf, w1x, w1n, wv1, b1, bv1)
    q = _prop1(adj, p, w2, wv2, b2, bv2)
    mm, ms, mz, lv3, z3, mv, zb = _prop2(adj, q, eps_u)

    amm, ams, amz, alv3, za3, amv, zab = _attr(
        x, nfa, Wam1[NOISE:], Wam1[:NOISE],
        Wav1, bam1.reshape(1, H), bav1.reshape(1, H),
        Wam2, Wav2, bam2.reshape(1, OUT),
        bav2.reshape(1, OUT), eps_a)

    lu, la = _decode(zb, zab)

    return (
        mm,
        ms,
        mz,
        lv3,
        z3,
        amm,
        ams,
        amz,
        alv3,
        za3,
        lu.reshape(n, n, 1),
        la.reshape(n, a, 1),
        mv,
        amv,
    )


# speedup vs baseline: 1.0103x; 1.0103x over previous
import functools

import jax
import jax.numpy as jnp
import numpy as np
from jax.experimental import pallas as pl

N = 4096
D = 512
H = 128
OUT = 512
NOISE = 5

_BLK = 256
_BLK2 = 256
_PRE_BLK = 512

_bf16 = jnp.bfloat16
_f32 = jnp.float32


def _dot(a, b):
    return jax.lax.dot_general(
        a, b, (((1,), (0,)), ((), ())), preferred_element_type=_f32
    )


_U_LO = np.float32(-0.99999994)
_U_SPAN = np.float32(1.99999994)
_SQRT2 = np.float32(1.4142135623730951)


def _normal_from_bits(bits):
    u01 = jax.lax.bitcast_convert_type(
        (bits >> 9) | np.uint32(0x3F800000), _f32) - 1.0
    u = jnp.maximum(_U_LO, u01 * _U_SPAN + _U_LO)
    w = -jnp.log1p(-u * u)
    ws = w - 2.5
    ps = np.float32(2.81022636e-08)
    for c in (3.43273939e-07, -3.5233877e-06, -4.39150654e-06, 0.00021858087,
              -0.00125372503, -0.00417768164, 0.246640727, 1.50140941):
        ps = np.float32(c) + ps * ws
    wb = jnp.sqrt(w) - 3.0
    pb = np.float32(-0.000200214257)
    for c in (0.000100950558, 0.00134934322, -0.00367342844, 0.00573950773,
              -0.0076224613, 0.00943887047, 1.00167406, 2.83297682):
        pb = np.float32(c) + pb * wb
    return _SQRT2 * jnp.where(w < 5.0, ps, pb) * u


def _pre_kernel(x_ref, nf_ref, w1x_ref, w1n_ref, wv1_ref, b1_ref, bv1_ref, p_ref):
    xb = x_ref[...].astype(_bf16)
    w1n = w1n_ref[...].astype(_bf16)
    g_mu = _dot(xb, w1x_ref[...].astype(_bf16))
    g_lv = _dot(xb, wv1_ref[...].astype(_bf16))
    n0 = _dot(nf_ref[:, 0:NOISE].astype(_bf16), w1n)
    n1 = _dot(nf_ref[:, NOISE:2 * NOISE].astype(_bf16), w1n)
    b1 = b1_ref[...]
    bv1 = bv1_ref[...]
    p0 = g_mu + n0 + b1
    p1 = g_mu + n1 + b1
    pv = g_lv + bv1
    p_ref[...] = jnp.concatenate([p0, p1, pv], axis=1).astype(_bf16)


def _pre(x, nf, w1x, w1n, wv1, b1, bv1):
    grid = N // _PRE_BLK
    return pl.pallas_call(
        _pre_kernel,
        grid=(grid,),
        in_specs=[
            pl.BlockSpec((_PRE_BLK, D), lambda i: (i, 0)),
            pl.BlockSpec((_PRE_BLK, 2 * NOISE), lambda i: (i, 0)),
            pl.BlockSpec((D, H), lambda i: (0, 0)),
            pl.BlockSpec((NOISE, H), lambda i: (0, 0)),
            pl.BlockSpec((D, H), lambda i: (0, 0)),
            pl.BlockSpec((1, H), lambda i: (0, 0)),
            pl.BlockSpec((1, H), lambda i: (0, 0)),
        ],
        out_specs=pl.BlockSpec((_PRE_BLK, 3 * H), lambda i: (i, 0)),
        out_shape=jax.ShapeDtypeStruct((N, 3 * H), _bf16),
    )(x, nf, w1x, w1n, wv1, b1, bv1)


def _prop1_kernel(adj_ref, p_ref, w2_ref, wv2_ref, b2_ref, bv2_ref, q_ref):
    t1 = _dot(adj_ref[...].astype(_bf16), p_ref[...])
    h = jax.nn.relu(t1).astype(_bf16)
    w2 = w2_ref[...].astype(_bf16)
    q0 = _dot(h[:, 0:H], w2) + b2_ref[...]
    q1 = _dot(h[:, H:2 * H], w2) + b2_ref[...]
    qv = _dot(h[:, 2 * H:3 * H], wv2_ref[...].astype(_bf16)) + bv2_ref[...]
    q_ref[...] = jnp.concatenate([q0, q1, qv], axis=1).astype(_bf16)


def _prop1(adj, p, w2, wv2, b2, bv2):
    grid = N // _BLK
    return pl.pallas_call(
        _prop1_kernel,
        grid=(grid,),
        in_specs=[
            pl.BlockSpec((_BLK, N), lambda i: (i, 0)),
            pl.BlockSpec((N, 3 * H), lambda i: (0, 0)),
            pl.BlockSpec((H, OUT), lambda i: (0, 0)),
            pl.BlockSpec((H, OUT), lambda i: (0, 0)),
            pl.BlockSpec((1, OUT), lambda i: (0, 0)),
            pl.BlockSpec((1, OUT), lambda i: (0, 0)),
        ],
        out_specs=pl.BlockSpec((_BLK, 3 * OUT), lambda i: (i, 0)),
        out_shape=jax.ShapeDtypeStruct((N, 3 * OUT), _bf16),
    )(adj, p, w2, wv2, b2, bv2)


def _prop2_kernel(adj_ref, q_ref, eps_ref, mm_ref, ms_ref, mz_ref,
                  lv_ref, z_ref, mv_ref, zb_ref):
    t2 = _dot(adj_ref[...].astype(_bf16), q_ref[...])
    mu_iw = t2[:, 0:OUT]
    mu_star = t2[:, OUT:2 * OUT]
    logv = t2[:, 2 * OUT:3 * OUT]
    sigma = jnp.exp(0.5 * logv)
    z = mu_iw + sigma * _normal_from_bits(eps_ref[...])
    mm_ref[:, 0, :] = mu_star
    mm_ref[:, 1, :] = mu_iw
    ms_ref[:, 0, :] = sigma
    ms_ref[:, 1, :] = sigma
    mz_ref[:, 0, :] = z
    mz_ref[:, 1, :] = z
    lv_ref[...] = logv
    z_ref[...] = z
    mv_ref[...] = mu_iw
    zb_ref[...] = z.astype(_bf16)


def _prop2(adj, q, eps):
    grid = N // _BLK2
    wide = jax.ShapeDtypeStruct((N, 1, 2, OUT), _f32)
    slim3 = jax.ShapeDtypeStruct((N, 1, OUT), _f32)
    wspec = pl.BlockSpec((_BLK2, None, 2, OUT), lambda i: (i, 0, 0, 0))
    sspec3 = pl.BlockSpec((_BLK2, None, OUT), lambda i: (i, 0, 0))
    sspec = pl.BlockSpec((_BLK2, OUT), lambda i: (i, 0))
    return pl.pallas_call(
        _prop2_kernel,
        grid=(grid,),
        in_specs=[
            pl.BlockSpec((_BLK2, N), lambda i: (i, 0)),
            pl.BlockSpec((N, 3 * OUT), lambda i: (0, 0)),
            sspec,
        ],
        out_specs=[wspec, wspec, wspec, sspec3, sspec3, sspec, sspec],
        out_shape=[wide, wide, wide, slim3, slim3,
                   jax.ShapeDtypeStruct((N, OUT), _f32),
                   jax.ShapeDtypeStruct((N, OUT), _bf16)],
    )(adj, q, eps)


def _attr_kernel(x_ref, nfa_ref, wax_ref, wan_ref, wavx_ref, ba1_ref, bav1_ref,
                 wa2_ref, wav2_ref, ba2_ref, bav2_ref, epsa_ref,
                 mm_ref, ms_ref, mz_ref, lv_ref, z_ref, mv_ref, zb_ref):
    xb = x_ref[...].astype(_bf16)
    tdot = lambda a, b: jax.lax.dot_general(
        a, b, (((0,), (0,)), ((), ())), preferred_element_type=_f32)
    wan = wan_ref[...].astype(_bf16)
    g = tdot(xb, wax_ref[...].astype(_bf16))
    gv = tdot(xb, wavx_ref[...].astype(_bf16))
    n0 = _dot(nfa_ref[:, 0:NOISE].astype(_bf16), wan)
    n1 = _dot(nfa_ref[:, NOISE:2 * NOISE].astype(_bf16), wan)
    ba1 = ba1_ref[...]
    h0 = jax.nn.relu(g + n0 + ba1).astype(_bf16)
    h1 = jax.nn.relu(g + n1 + ba1).astype(_bf16)
    hv = jax.nn.relu(gv + bav1_ref[...]).astype(_bf16)
    wa2 = wa2_ref[...].astype(_bf16)
    mu_iw = _dot(h0, wa2) + ba2_ref[...]
    mu_star = _dot(h1, wa2) + ba2_ref[...]
    logv = _dot(hv, wav2_ref[...].astype(_bf16)) + bav2_ref[...]
    sigma = jnp.exp(0.5 * logv)
    z = mu_iw + sigma * _normal_from_bits(epsa_ref[...])
    mm_ref[:, 0, :] = mu_star
    mm_ref[:, 1, :] = mu_iw
    ms_ref[:, 0, :] = sigma
    ms_ref[:, 1, :] = sigma
    mz_ref[:, 0, :] = z
    mz_ref[:, 1, :] = z
    lv_ref[...] = logv
    z_ref[...] = z
    mv_ref[...] = mu_iw
    zb_ref[...] = z.astype(_bf16)


def _attr(x, nfa, wax, wan, wavx, ba1, bav1, wa2, wav2, ba2, bav2, epsa):
    a = D
    full = lambda r, c: pl.BlockSpec((r, c), lambda i: (0, 0))
    full3 = lambda r, c: pl.BlockSpec((r, None, c), lambda i: (0, 0, 0))
    full4 = lambda r, c: pl.BlockSpec((r, None, 2, c), lambda i: (0, 0, 0, 0))
    wide = jax.ShapeDtypeStruct((a, 1, 2, OUT), _f32)
    slim3 = jax.ShapeDtypeStruct((a, 1, OUT), _f32)
    return pl.pallas_call(
        _attr_kernel,
        grid=(1,),
        in_specs=[
            full(N, a), full(a, 2 * NOISE), full(N, H), full(NOISE, H),
            full(N, H), full(1, H), full(1, H), full(H, OUT), full(H, OUT),
            full(1, OUT), full(1, OUT), full(a, OUT),
        ],
        out_specs=[full4(a, OUT), full4(a, OUT), full4(a, OUT),
                   full3(a, OUT), full3(a, OUT), full(a, OUT), full(a, OUT)],
        out_shape=[wide, wide, wide, slim3, slim3,
                   jax.ShapeDtypeStruct((a, OUT), _f32),
                   jax.ShapeDtypeStruct((a, OUT), _bf16)],
    )(x, nfa, wax, wan, wavx, ba1, bav1, wa2, wav2, ba2, bav2, epsa)


def _decode_kernel(zu_ref, zall_ref, za_ref, lu_ref, la_ref):
    zu = zu_ref[...]
    bt_dot = lambda a, b: jax.lax.dot_general(
        a, b, (((1,), (1,)), ((), ())), preferred_element_type=_f32)
    lu_ref[...] = bt_dot(zu, zall_ref[...])
    la_ref[...] = bt_dot(zu, za_ref[...])


def _decode(zu_bf, za_bf):
    grid = N // _BLK2
    return pl.pallas_call(
        _decode_kernel,
        grid=(grid,),
        in_specs=[
            pl.BlockSpec((_BLK2, OUT), lambda i: (i, 0)),
            pl.BlockSpec((N, OUT), lambda i: (0, 0)),
            pl.BlockSpec((D, OUT), lambda i: (0, 0)),
        ],
        out_specs=[
            pl.BlockSpec((_BLK2, None, N), lambda i: (i, 0, 0)),
            pl.BlockSpec((_BLK2, None, D), lambda i: (i, 0, 0)),
        ],
        out_shape=[
            jax.ShapeDtypeStruct((N, 1, N), _f32),
            jax.ShapeDtypeStruct((N, 1, D), _f32),
        ],
    )(zu_bf, zu_bf, za_bf)


def kernel(adj, x, Wnm1, bnm1, Wnm2, bnm2, Wnv1, bnv1, Wnv2, bnv2,
           Wam1, bam1, Wam2, bam2, Wav1, bav1, Wav2, bav2):
    n, a = x.shape
    key = jax.random.key(42)
    k1, k2, k3, k4 = jax.random.split(key, 4)
    nf = jax.random.bernoulli(k1, 0.5, (n, 2 * NOISE)).astype(_f32)
    nfa = jax.random.bernoulli(k3, 0.5, (a, 2 * NOISE)).astype(_f32)
    eps_u = jax.random.bits(k2, (n, OUT), jnp.uint32)
    eps_a = jax.random.bits(k4, (a, OUT), jnp.uint32)

    w1x = Wnm1[NOISE:]
    w1n = Wnm1[:NOISE]
    wv1 = Wnv1
    w2 = Wnm2
    wv2 = Wnv2
    b1 = bnm1.reshape(1, H)
    bv1 = bnv1.reshape(1, H)
    b2 = bnm2.reshape(1, OUT)
    bv2 = bnv2.reshape(1, OUT)

    p = _pre(x, nf, w1x, w1n, wv1, b1, bv1)
    q = _prop1(adj, p, w2, wv2, b2, bv2)
    mm, ms, mz, lv3, z3, mv, zb = _prop2(adj, q, eps_u)

    amm, ams, amz, alv3, za3, amv, zab = _attr(
        x, nfa, Wam1[NOISE:], Wam1[:NOISE],
        Wav1, bam1.reshape(1, H), bav1.reshape(1, H),
        Wam2, Wav2, bam2.reshape(1, OUT),
        bav2.reshape(1, OUT), eps_a)

    lu, la = _decode(zb, zab)

    return (
        mm,
        ms,
        mz,
        lv3,
        z3,
        amm,
        ams,
        amz,
        alv3,
        za3,
        lu.reshape(n, n, 1),
        la.reshape(n, a, 1),
        mv,
        amv,
    )
